# baseline (speedup 1.0000x reference)
import functools

import jax
import jax.numpy as jnp
from jax import lax
from jax.experimental import pallas as pl
from jax.experimental.pallas import tpu as pltpu
from jax.experimental.pallas import tpu_sc as plsc

VOCAB = 1000000
HIDDEN = 64
BATCH = 4096
SEQ = 200

NC = 2
NS = 16
NW = NC * NS

B = BATCH * SEQ
B_PER_W = B // NW
C = 128
N_CHUNKS = B_PER_W // C
NBUF = 8
N_ROUNDS = N_CHUNKS // NBUF

assert B_PER_W * NW == B and C * N_CHUNKS == B_PER_W and NBUF * N_ROUNDS == N_CHUNKS

_mesh = plsc.VectorSubcoreMesh(core_axis_name="c", subcore_axis_name="s")


@functools.partial(
    pl.kernel,
    out_type=jax.ShapeDtypeStruct((B, HIDDEN), jnp.float32),
    mesh=_mesh,
    scratch_types=[
        pltpu.VMEM((N_CHUNKS, C), jnp.int32),
        [pltpu.VMEM((C, HIDDEN), jnp.float32) for _ in range(NBUF)],
        [pltpu.SemaphoreType.DMA for _ in range(NBUF)],
        [pltpu.SemaphoreType.DMA for _ in range(NBUF)],
    ],
    compiler_params=pltpu.CompilerParams(use_tc_tiling_on_sc=False),
)
def _emb_gather(table_hbm, idx_hbm, out_hbm, idx_v, bufs, sem_g, sem_s):
    wid = lax.axis_index("s") * NC + lax.axis_index("c")
    base = wid * B_PER_W

    pltpu.sync_copy(idx_hbm.at[wid], idx_v)

    def start_gather(c, b):
        pltpu.make_async_copy(table_hbm.at[idx_v.at[c]], bufs[b], sem_g[b]).start()

    def wait_gather(b):
        pltpu.make_async_copy(table_hbm.at[idx_v.at[0]], bufs[b], sem_g[b]).wait()

    def start_scatter(c, b):
        pltpu.make_async_copy(
            bufs[b], out_hbm.at[pl.ds(base + c * C, C)], sem_s[b]
        ).start()

    def wait_scatter(b):
        pltpu.make_async_copy(
            bufs[b], out_hbm.at[pl.ds(base, C)], sem_s[b]
        ).wait()

    for b in range(NBUF):
        start_gather(b, b)

    def round_body(r, _):
        for b in range(NBUF):
            c = r * NBUF + b
            wait_gather(b)
            start_scatter(c, b)
            wait_scatter(b)
            start_gather(c + NBUF, b)
        return _

    lax.fori_loop(0, N_ROUNDS - 1, round_body, 0, unroll=False)

    for b in range(NBUF):
        c = (N_ROUNDS - 1) * NBUF + b
        wait_gather(b)
        start_scatter(c, b)
    for b in range(NBUF):
        wait_scatter(b)


def kernel(input_ids, emb):
    ids = input_ids.reshape(-1).astype(jnp.int32)
    idx3 = ids.reshape(NW, N_CHUNKS, C)
    out = _emb_gather(emb, idx3)
    return out.reshape(BATCH, SEQ, HIDDEN)
